# baseline (speedup 1.0000x reference)
import jax
import jax.numpy as jnp
from jax.experimental import pallas as pl
from jax.experimental.pallas import tpu as pltpu

B, N = 16, 24
NODE_F, EDGE_F = 64, 4
HIDDEN = 128
MSG = 128
PASSES = 3
F_ADD = 40
GATHER_W = 128
BIG_POSITIVE = 1e6

SPLIT = 1


def _ggnn_body(h0_ref, nodes_ref, ev_ref, *rest):
    refs = list(rest)
    out_fadd2, out_fconn2, out_fterm2 = refs[-3], refs[-2], refs[-1]
    it = iter(refs[:-3])

    def nxt():
        return next(it)[...]

    msg = [[(nxt(), nxt()) for _layer in range(3)] for _f in range(EDGE_F)]
    gWih, gWhh, gbih, gbhh = nxt(), nxt(), nxt(), nxt()
    aWh, aWn, ab1, aW2, ab2, aW3, ab3 = (nxt() for _ in range(7))
    eW1, eb1, eW2, eb2, eW3, eb3 = (nxt() for _ in range(6))
    fa1W1, fa1b1, fa1W2, fa1b2, fa1W3, fa1b3 = (nxt() for _ in range(6))
    fc1W1, fc1b1, fc1W2, fc1b2, fc1W3, fc1b3 = (nxt() for _ in range(6))
    fa2Wf, fa2Wg, fa2b1, fa2W2, fa2b2, fa2W3, fa2b3 = (nxt() for _ in range(7))
    fc2Wf, fc2Wg, fc2b1, fc2W2, fc2b2, fc2W3, fc2b3 = (nxt() for _ in range(7))
    ftW1, ftb1, ftW2, ftb2, ftW3, ftb3 = (nxt() for _ in range(6))

    A = 1.6732632423543772
    LOG2E = 1.4426950408889634
    LOG2A = 0.7427357821433388

    def selu_u(x):
        return jnp.where(x > 0, x, jnp.exp2(x * LOG2E + LOG2A) - A)

    relu = lambda x: jnp.maximum(x, 0.0)

    h = h0_ref[...]
    nodes2 = nodes_ref[...]
    ev = ev_ref[...]
    BB = h.shape[0] // N
    EE = BB * N * N

    bf = jnp.bfloat16
    ev_bf = ev.astype(bf)
    for _p in range(PASSES):
        cs = []
        for f in range(EDGE_F):
            (W1, b1), (W2, b2), (W3, b3) = msg[f]
            g1 = jnp.dot(h, W1).astype(bf)
            g1t = jnp.broadcast_to(
                g1.reshape(BB, 1, N, HIDDEN), (BB, N, N, HIDDEN)
            ).reshape(EE, HIDDEN)
            y = selu_u(ev_bf[:, f:f + 1] * g1t + b1.astype(bf))
            y = selu_u(jnp.dot(y, W2.astype(bf),
                               preferred_element_type=jnp.float32
                               ).astype(bf) + b2.astype(bf))
            y = selu_u(jnp.dot(y, W3.astype(bf),
                               preferred_element_type=jnp.float32
                               ).astype(bf) + b3.astype(bf))
            cs.append(ev_bf[:, f:f + 1] * y)
        t = (cs[0] + cs[1]) + (cs[2] + cs[3])
        messages = jnp.sum(t.reshape(BB * N, N, MSG),
                           axis=1).astype(jnp.float32)
        gi = jnp.dot(messages, gWih) + gbih
        gh = jnp.dot(h, gWhh) + gbhh
        r = jax.nn.sigmoid(gi[:, :HIDDEN] + gh[:, :HIDDEN])
        z = jax.nn.sigmoid(gi[:, HIDDEN:2 * HIDDEN] + gh[:, HIDDEN:2 * HIDDEN])
        n = jnp.tanh(gi[:, 2 * HIDDEN:] + r * gh[:, 2 * HIDDEN:])
        h = (1.0 - z) * n + z * h

    rowsum = jnp.sum(ev.reshape(BB * N, N, EDGE_F), axis=(1, 2)).reshape(BB * N, 1)
    pen = jnp.where(rowsum != 0.0, 0.0, BIG_POSITIVE)
    a = selu_u(jnp.dot(h, aWh) + jnp.dot(nodes2, aWn) + ab1)
    a = selu_u(jnp.dot(a, aW2) + ab2)
    energies = 1.0507009873554805 * selu_u(jnp.dot(a, aW3) + ab3) - pen
    e3 = energies.reshape(BB, N, GATHER_W)
    e3 = e3 - jnp.max(e3, axis=1, keepdims=True)
    ex = jnp.exp(e3)
    attn = ex / jnp.sum(ex, axis=1, keepdims=True)
    emb = selu_u(jnp.dot(h, eW1) + eb1)
    emb = selu_u(jnp.dot(emb, eW2) + eb2)
    emb = selu_u(jnp.dot(emb, eW3) + eb3)
    gph = jnp.sum(attn * emb.reshape(BB, N, GATHER_W), axis=1)

    fa = relu(jnp.dot(h, fa1W1) + fa1b1)
    fa = relu(jnp.dot(fa, fa1W2) + fa1b2)
    fa = relu(jnp.dot(fa, fa1W3) + fa1b3)
    fc = relu(jnp.dot(h, fc1W1) + fc1b1)
    fc = relu(jnp.dot(fc, fc1W2) + fc1b2)
    fc = relu(jnp.dot(fc, fc1W3) + fc1b3)

    fa3 = fa.reshape(BB, N, F_ADD)
    acc = jnp.dot(gph, fa2Wg) + fa2b1
    for nn in range(N):
        acc = acc + jnp.dot(fa3[:, nn, :], fa2Wf[nn])
    x = relu(acc)
    x = relu(jnp.dot(x, fa2W2) + fa2b2)
    out_fadd2[...] = relu(jnp.dot(x, fa2W3) + fa2b3)

    fc3 = fc.reshape(BB, N, EDGE_F)
    acc = jnp.dot(gph, fc2Wg) + fc2b1
    for nn in range(N):
        acc = acc + jnp.dot(fc3[:, nn, :], fc2Wf[nn])
    x = relu(acc)
    x = relu(jnp.dot(x, fc2W2) + fc2b2)
    out_fconn2[...] = relu(jnp.dot(x, fc2W3) + fc2b3)

    x = relu(jnp.dot(gph, ftW1) + ftb1)
    x = relu(jnp.dot(x, ftW2) + ftb2)
    out_fterm2[...] = relu(jnp.dot(x, ftW3) + ftb3)


def _data_spec(shape):
    blk = (shape[0] // SPLIT,) + tuple(shape[1:])
    nd = len(shape)
    return pl.BlockSpec(blk, lambda c, _nd=nd: (c,) + (0,) * (_nd - 1))


def _weight_spec(shape):
    nd = len(shape)
    return pl.BlockSpec(tuple(shape), lambda c, _nd=nd: (0,) * _nd)


def kernel(nodes, edges, params):
    nodes2 = nodes.reshape(B * N, NODE_F).astype(jnp.float32)
    h0 = jnp.concatenate(
        [nodes2, jnp.zeros((B * N, HIDDEN - NODE_F), jnp.float32)], axis=1)
    ev = edges.reshape(B * N * N, EDGE_F).astype(jnp.float32)

    S = 1.0507009873554805

    args = [h0, nodes2, ev]
    n_data = len(args)
    for f in range(EDGE_F):
        (W1, b1), (W2, b2), (W3, b3) = params['msg'][f]
        args += [W1.T, b1.reshape(1, -1),
                 S * W2.T, b2.reshape(1, -1),
                 S * W3.T, b3.reshape(1, -1)]
    Wih, Whh, bih, bhh = params['gru']
    args += [S * Wih.T, Whh.T, bih.reshape(1, -1), bhh.reshape(1, -1)]
    (aW1, ab1), (aW2, ab2), (aW3, ab3) = params['att']
    args += [aW1[:, :HIDDEN].T, aW1[:, HIDDEN:].T, ab1.reshape(1, -1),
             S * aW2.T, ab2.reshape(1, -1), S * aW3.T, ab3.reshape(1, -1)]
    (eW1, eb1), (eW2, eb2), (eW3, eb3) = params['emb']
    args += [eW1.T, eb1.reshape(1, -1), S * eW2.T, eb2.reshape(1, -1),
             S * eW3.T, eb3.reshape(1, -1)]
    for name in ('fadd1', 'fconn1'):
        for (W, b) in params[name]:
            args += [W.T, b.reshape(1, -1)]
    for name, c in (('fadd2', F_ADD), ('fconn2', EDGE_F)):
        (W1, b1), (W2, b2), (W3, b3) = params[name]
        Wf = W1[:, :N * c].T.reshape(N, c, W1.shape[0])
        Wg = S * W1[:, N * c:].T
        args += [Wf, Wg, b1.reshape(1, -1),
                 W2.T, b2.reshape(1, -1), W3.T, b3.reshape(1, -1)]
    (tW1, tb1), (tW2, tb2), (tW3, tb3) = params['fterm2']
    args += [S * tW1.T, tb1.reshape(1, -1), tW2.T, tb2.reshape(1, -1),
             tW3.T, tb3.reshape(1, -1)]

    in_specs = [_data_spec(a.shape) for a in args[:n_data]]
    in_specs += [_weight_spec(a.shape) for a in args[n_data:]]
    out_shapes = (
        jax.ShapeDtypeStruct((B, N * F_ADD), jnp.float32),
        jax.ShapeDtypeStruct((B, N * EDGE_F), jnp.float32),
        jax.ShapeDtypeStruct((B, 1), jnp.float32),
    )
    out_specs = tuple(_data_spec(s.shape) for s in out_shapes)

    fadd2, fconn2, fterm2 = pl.pallas_call(
        _ggnn_body,
        grid=(SPLIT,),
        in_specs=in_specs,
        out_specs=out_specs,
        out_shape=out_shapes,
        compiler_params=pltpu.CompilerParams(
            dimension_semantics=("parallel",)),
    )(*args)
    return jnp.concatenate([fadd2, fconn2, fterm2], axis=1)


# speedup vs baseline: 1.0040x; 1.0040x over previous
import jax
import jax.numpy as jnp
from jax.experimental import pallas as pl
from jax.experimental.pallas import tpu as pltpu

B, N = 16, 24
NODE_F, EDGE_F = 64, 4
HIDDEN = 128
MSG = 128
PASSES = 3
F_ADD = 40
GATHER_W = 128
BIG_POSITIVE = 1e6

SPLIT = 1


def _ggnn_body(h0_ref, nodes_ref, ev_ref, *rest):
    refs = list(rest)
    out_fadd2, out_fconn2, out_fterm2 = refs[-3], refs[-2], refs[-1]
    it = iter(refs[:-3])

    def nxt():
        return next(it)[...]

    msg = [[(nxt(), nxt()) for _layer in range(3)] for _f in range(EDGE_F)]
    gWih, gWhh, gbih, gbhh = nxt(), nxt(), nxt(), nxt()
    aWh, aWn, ab1, aW2, ab2, aW3, ab3 = (nxt() for _ in range(7))
    eW1, eb1, eW2, eb2, eW3, eb3 = (nxt() for _ in range(6))
    fa1W1, fa1b1, fa1W2, fa1b2, fa1W3, fa1b3 = (nxt() for _ in range(6))
    fc1W1, fc1b1, fc1W2, fc1b2, fc1W3, fc1b3 = (nxt() for _ in range(6))
    fa2Wf, fa2Wg, fa2b1, fa2W2, fa2b2, fa2W3, fa2b3 = (nxt() for _ in range(7))
    fc2Wf, fc2Wg, fc2b1, fc2W2, fc2b2, fc2W3, fc2b3 = (nxt() for _ in range(7))
    ftW1, ftb1, ftW2, ftb2, ftW3, ftb3 = (nxt() for _ in range(6))

    A = 1.6732632423543772
    LOG2E = 1.4426950408889634
    LOG2A = 0.7427357821433388

    def selu_u(x):
        return jnp.where(x > 0, x, jnp.exp2(x * LOG2E + LOG2A) - A)

    relu = lambda x: jnp.maximum(x, 0.0)

    h = h0_ref[...]
    nodes2 = nodes_ref[...]
    ev = ev_ref[...]
    BB = h.shape[0] // N
    EE = BB * N * N

    bf = jnp.bfloat16
    ev_bf = ev.astype(bf)
    for _p in range(PASSES):
        cs = []
        for f in range(EDGE_F):
            (W1, b1), (W2, b2), (W3, b3) = msg[f]
            g1 = jnp.dot(h, W1).astype(bf)
            g1t = jnp.broadcast_to(
                g1.reshape(BB, 1, N, HIDDEN), (BB, N, N, HIDDEN)
            ).reshape(EE, HIDDEN)
            y = selu_u(ev_bf[:, f:f + 1] * g1t + b1.astype(bf))
            y = selu_u(jnp.dot(y, W2.astype(bf),
                               preferred_element_type=jnp.float32
                               ).astype(bf) + b2.astype(bf))
            y = selu_u(jnp.dot(y, W3.astype(bf),
                               preferred_element_type=jnp.float32
                               ).astype(bf) + b3.astype(bf))
            cs.append(ev_bf[:, f:f + 1] * y)
        t = (cs[0] + cs[1]) + (cs[2] + cs[3])
        messages = jnp.sum(
            t.reshape(BB * N, N, MSG).astype(jnp.float32), axis=1)
        gi = jnp.dot(messages, gWih) + gbih
        gh = jnp.dot(h, gWhh) + gbhh
        r = jax.nn.sigmoid(gi[:, :HIDDEN] + gh[:, :HIDDEN])
        z = jax.nn.sigmoid(gi[:, HIDDEN:2 * HIDDEN] + gh[:, HIDDEN:2 * HIDDEN])
        n = jnp.tanh(gi[:, 2 * HIDDEN:] + r * gh[:, 2 * HIDDEN:])
        h = (1.0 - z) * n + z * h

    rowsum = jnp.sum(ev.reshape(BB * N, N, EDGE_F), axis=(1, 2)).reshape(BB * N, 1)
    pen = jnp.where(rowsum != 0.0, 0.0, BIG_POSITIVE)
    a = selu_u(jnp.dot(h, aWh) + jnp.dot(nodes2, aWn) + ab1)
    a = selu_u(jnp.dot(a, aW2) + ab2)
    energies = 1.0507009873554805 * selu_u(jnp.dot(a, aW3) + ab3) - pen
    e3 = energies.reshape(BB, N, GATHER_W)
    e3 = e3 - jnp.max(e3, axis=1, keepdims=True)
    ex = jnp.exp(e3)
    attn = ex / jnp.sum(ex, axis=1, keepdims=True)
    emb = selu_u(jnp.dot(h, eW1) + eb1)
    emb = selu_u(jnp.dot(emb, eW2) + eb2)
    emb = selu_u(jnp.dot(emb, eW3) + eb3)
    gph = jnp.sum(attn * emb.reshape(BB, N, GATHER_W), axis=1)

    fa = relu(jnp.dot(h, fa1W1) + fa1b1)
    fa = relu(jnp.dot(fa, fa1W2) + fa1b2)
    fa = relu(jnp.dot(fa, fa1W3) + fa1b3)
    fc = relu(jnp.dot(h, fc1W1) + fc1b1)
    fc = relu(jnp.dot(fc, fc1W2) + fc1b2)
    fc = relu(jnp.dot(fc, fc1W3) + fc1b3)

    fa3 = fa.reshape(BB, N, F_ADD)
    acc = jnp.dot(gph, fa2Wg) + fa2b1
    for nn in range(N):
        acc = acc + jnp.dot(fa3[:, nn, :], fa2Wf[nn])
    x = relu(acc)
    x = relu(jnp.dot(x, fa2W2) + fa2b2)
    out_fadd2[...] = relu(jnp.dot(x, fa2W3) + fa2b3)

    fc3 = fc.reshape(BB, N, EDGE_F)
    acc = jnp.dot(gph, fc2Wg) + fc2b1
    for nn in range(N):
        acc = acc + jnp.dot(fc3[:, nn, :], fc2Wf[nn])
    x = relu(acc)
    x = relu(jnp.dot(x, fc2W2) + fc2b2)
    out_fconn2[...] = relu(jnp.dot(x, fc2W3) + fc2b3)

    x = relu(jnp.dot(gph, ftW1) + ftb1)
    x = relu(jnp.dot(x, ftW2) + ftb2)
    out_fterm2[...] = relu(jnp.dot(x, ftW3) + ftb3)


def _data_spec(shape):
    blk = (shape[0] // SPLIT,) + tuple(shape[1:])
    nd = len(shape)
    return pl.BlockSpec(blk, lambda c, _nd=nd: (c,) + (0,) * (_nd - 1))


def _weight_spec(shape):
    nd = len(shape)
    return pl.BlockSpec(tuple(shape), lambda c, _nd=nd: (0,) * _nd)


def kernel(nodes, edges, params):
    nodes2 = nodes.reshape(B * N, NODE_F).astype(jnp.float32)
    h0 = jnp.concatenate(
        [nodes2, jnp.zeros((B * N, HIDDEN - NODE_F), jnp.float32)], axis=1)
    ev = edges.reshape(B * N * N, EDGE_F).astype(jnp.float32)

    S = 1.0507009873554805

    args = [h0, nodes2, ev]
    n_data = len(args)
    for f in range(EDGE_F):
        (W1, b1), (W2, b2), (W3, b3) = params['msg'][f]
        args += [W1.T, b1.reshape(1, -1),
                 S * W2.T, b2.reshape(1, -1),
                 S * W3.T, b3.reshape(1, -1)]
    Wih, Whh, bih, bhh = params['gru']
    args += [S * Wih.T, Whh.T, bih.reshape(1, -1), bhh.reshape(1, -1)]
    (aW1, ab1), (aW2, ab2), (aW3, ab3) = params['att']
    args += [aW1[:, :HIDDEN].T, aW1[:, HIDDEN:].T, ab1.reshape(1, -1),
             S * aW2.T, ab2.reshape(1, -1), S * aW3.T, ab3.reshape(1, -1)]
    (eW1, eb1), (eW2, eb2), (eW3, eb3) = params['emb']
    args += [eW1.T, eb1.reshape(1, -1), S * eW2.T, eb2.reshape(1, -1),
             S * eW3.T, eb3.reshape(1, -1)]
    for name in ('fadd1', 'fconn1'):
        for (W, b) in params[name]:
            args += [W.T, b.reshape(1, -1)]
    for name, c in (('fadd2', F_ADD), ('fconn2', EDGE_F)):
        (W1, b1), (W2, b2), (W3, b3) = params[name]
        Wf = W1[:, :N * c].T.reshape(N, c, W1.shape[0])
        Wg = S * W1[:, N * c:].T
        args += [Wf, Wg, b1.reshape(1, -1),
                 W2.T, b2.reshape(1, -1), W3.T, b3.reshape(1, -1)]
    (tW1, tb1), (tW2, tb2), (tW3, tb3) = params['fterm2']
    args += [S * tW1.T, tb1.reshape(1, -1), tW2.T, tb2.reshape(1, -1),
             tW3.T, tb3.reshape(1, -1)]

    in_specs = [_data_spec(a.shape) for a in args[:n_data]]
    in_specs += [_weight_spec(a.shape) for a in args[n_data:]]
    out_shapes = (
        jax.ShapeDtypeStruct((B, N * F_ADD), jnp.float32),
        jax.ShapeDtypeStruct((B, N * EDGE_F), jnp.float32),
        jax.ShapeDtypeStruct((B, 1), jnp.float32),
    )
    out_specs = tuple(_data_spec(s.shape) for s in out_shapes)

    fadd2, fconn2, fterm2 = pl.pallas_call(
        _ggnn_body,
        grid=(SPLIT,),
        in_specs=in_specs,
        out_specs=out_specs,
        out_shape=out_shapes,
        compiler_params=pltpu.CompilerParams(
            dimension_semantics=("parallel",)),
    )(*args)
    return jnp.concatenate([fadd2, fconn2, fterm2], axis=1)
